# baseline (speedup 1.0000x reference)
import functools

import jax
import jax.numpy as jnp
from jax import lax
from jax.experimental import pallas as pl
from jax.experimental.pallas import tpu as pltpu
from jax.experimental.pallas import tpu_sc as plsc

_B, _H, _N = 64, 16, 4096
_KS = (16, 64, 256, 1024)
_ROWS = _B * _H
_BLK = 128
_SC_ROWS = 512
_TC_ROWS = _ROWS - _SC_ROWS
_NW = 32
_RPW = _SC_ROWS // _NW
_I32_MAX = 0x7FFFFFFF
_I32_MIN = -(2**31)


def _gumbel_const():
    u = jax.random.uniform(jax.random.key(42), (_B, _H, _N), dtype=jnp.float32)
    g = -jnp.log(-jnp.log(u + 1e-20) + 1e-20)
    return g.reshape(_ROWS, _N)


_GUMBEL = _gumbel_const()


def _key_of(p):
    bits = lax.bitcast_convert_type(p, jnp.int32)
    return jnp.where(bits < 0, bits ^ _I32_MAX, bits)


def _topk_mask_kernel(s_ref, g_ref, o_ref):
    key = _key_of(s_ref[...] + g_ref[...])
    r = key.shape[0]
    masks = []
    for k in _KS:
        t = jnp.full((r, 1), _I32_MIN, dtype=jnp.int32)
        cand = t & _I32_MAX
        cnt = jnp.sum((key >= cand).astype(jnp.int32), axis=1, keepdims=True)
        t = jnp.where(cnt >= k, cand, t)
        for b in range(30, -1, -1):
            cand = t | jnp.int32(1 << b)
            cnt = jnp.sum((key >= cand).astype(jnp.int32), axis=1, keepdims=True)
            t = jnp.where(cnt >= k, cand, t)
        masks.append((key >= t).astype(jnp.float32))
    o_ref[...] = jnp.stack(masks, axis=1)


def _mask_from_t_kernel(s_ref, g_ref, t_ref, o_ref):
    key = _key_of(s_ref[...] + g_ref[...])
    masks = []
    for j in range(len(_KS)):
        masks.append((key >= t_ref[:, j:j + 1]).astype(jnp.float32))
    o_ref[...] = jnp.stack(masks, axis=1)


def _sc_select(s_hbm, g_hbm, out_hbm, s_v, g_v, key_v, t_v):
    wid = lax.axis_index("s") * 2 + lax.axis_index("c")
    lanes = lax.broadcasted_iota(jnp.int32, (16,), 0)

    def row_body(r, _):
        row = wid * _RPW + r
        pltpu.sync_copy(s_hbm.at[row], s_v)
        pltpu.sync_copy(g_hbm.at[row], g_v)

        def prep(j, _c):
            for u in range(8):
                off = j * 128 + u * 16
                sl = pl.ds(off, 16)
                key_v[sl] = _key_of(s_v[sl] + g_v[sl])
            return _c

        lax.fori_loop(0, 32, prep, 0)

        def count4(c0, c1, c2, c3):
            def body(j, accs):
                a0, a1, a2, a3 = accs
                for u in range(8):
                    kv = key_v[pl.ds(j * 128 + u * 16, 16)]
                    a0 = a0 + plsc.all_reduce_population_count(kv >= c0)
                    a1 = a1 + plsc.all_reduce_population_count(kv >= c1)
                    a2 = a2 + plsc.all_reduce_population_count(kv >= c2)
                    a3 = a3 + plsc.all_reduce_population_count(kv >= c3)
                return a0, a1, a2, a3

            z = jnp.zeros((16,), jnp.int32)
            return lax.fori_loop(0, 32, body, (z, z, z, z))

        zero = jnp.zeros((16,), jnp.int32)
        cnts = count4(zero, zero, zero, zero)
        ts = [jnp.where(cnts[i] >= _KS[i], zero, jnp.full((16,), _I32_MIN, jnp.int32))
              for i in range(4)]

        t0, t1, t2, t3 = ts
        for b in range(30, -1, -1):
            bit = 1 << b
            c0, c1, c2, c3 = t0 | bit, t1 | bit, t2 | bit, t3 | bit
            n0, n1, n2, n3 = count4(c0, c1, c2, c3)
            t0 = jnp.where(n0 >= _KS[0], c0, t0)
            t1 = jnp.where(n1 >= _KS[1], c1, t1)
            t2 = jnp.where(n2 >= _KS[2], c2, t2)
            t3 = jnp.where(n3 >= _KS[3], c3, t3)

        tv = jnp.where(lanes == 0, t0,
             jnp.where(lanes == 1, t1,
             jnp.where(lanes == 2, t2,
             jnp.where(lanes == 3, t3, zero))))
        t_v[pl.ds(r * 16, 16)] = tv
        return _

    lax.fori_loop(0, _RPW, row_body, 0)
    pltpu.sync_copy(t_v, out_hbm.at[pl.ds(wid * _RPW * 16, _RPW * 16)])


_sc_select_call = functools.partial(
    pl.kernel,
    out_type=jax.ShapeDtypeStruct((_SC_ROWS * 16,), jnp.int32),
    scratch_types=[
        pltpu.VMEM((_N,), jnp.float32),
        pltpu.VMEM((_N,), jnp.float32),
        pltpu.VMEM((_N,), jnp.int32),
        pltpu.VMEM((_RPW * 16,), jnp.int32),
    ],
    mesh=plsc.VectorSubcoreMesh(core_axis_name="c", subcore_axis_name="s"),
    compiler_params=pltpu.CompilerParams(needs_layout_passes=False),
)(_sc_select)


def kernel(scores):
    s2 = scores.reshape(_ROWS, _N)
    g2 = _GUMBEL

    s_tc, g_tc = s2[:_TC_ROWS], g2[:_TC_ROWS]
    s_sc, g_sc = s2[_TC_ROWS:], g2[_TC_ROWS:]

    t_flat = _sc_select_call(s_sc, g_sc)
    t_sc = t_flat.reshape(_SC_ROWS, 16)[:, :len(_KS)]

    out_tc = pl.pallas_call(
        _topk_mask_kernel,
        grid=(_TC_ROWS // _BLK,),
        in_specs=[
            pl.BlockSpec((_BLK, _N), lambda i: (i, 0)),
            pl.BlockSpec((_BLK, _N), lambda i: (i, 0)),
        ],
        out_specs=pl.BlockSpec((_BLK, len(_KS), _N), lambda i: (i, 0, 0)),
        out_shape=jax.ShapeDtypeStruct((_TC_ROWS, len(_KS), _N), jnp.float32),
    )(s_tc, g_tc)

    out_sc = pl.pallas_call(
        _mask_from_t_kernel,
        grid=(_SC_ROWS // _BLK,),
        in_specs=[
            pl.BlockSpec((_BLK, _N), lambda i: (i, 0)),
            pl.BlockSpec((_BLK, _N), lambda i: (i, 0)),
            pl.BlockSpec((_BLK, len(_KS)), lambda i: (i, 0)),
        ],
        out_specs=pl.BlockSpec((_BLK, len(_KS), _N), lambda i: (i, 0, 0)),
        out_shape=jax.ShapeDtypeStruct((_SC_ROWS, len(_KS), _N), jnp.float32),
    )(s_sc, g_sc, t_sc)

    out = jnp.concatenate([out_tc, out_sc], axis=0)
    return out.reshape(_B, _H, len(_KS), _N)


# speedup vs baseline: 1.7986x; 1.7986x over previous
import functools

import jax
import jax.numpy as jnp
from jax import lax
from jax.experimental import pallas as pl
from jax.experimental.pallas import tpu as pltpu
from jax.experimental.pallas import tpu_sc as plsc

_B, _H, _N = 64, 16, 4096
_KS = (16, 64, 256, 1024)
_ROWS = _B * _H
_BLK = 128
_SC_ROWS = 512
_TC_ROWS = _ROWS - _SC_ROWS
_NW = 32
_RPW = _SC_ROWS // _NW
_I32_MAX = 0x7FFFFFFF
_I32_MIN = -(2**31)


def _gumbel_const():
    u = jax.random.uniform(jax.random.key(42), (_B, _H, _N), dtype=jnp.float32)
    g = -jnp.log(-jnp.log(u + 1e-20) + 1e-20)
    return g.reshape(_ROWS, _N)


_GUMBEL = _gumbel_const()


def _key_of(p):
    bits = lax.bitcast_convert_type(p, jnp.int32)
    return jnp.where(bits < 0, bits ^ _I32_MAX, bits)


def _topk_mask_kernel(s_ref, g_ref, o_ref):
    key = _key_of(s_ref[...] + g_ref[...])
    r = key.shape[0]
    masks = []
    for k in _KS:
        t = jnp.full((r, 1), _I32_MIN, dtype=jnp.int32)
        cand = t & _I32_MAX
        cnt = jnp.sum((key >= cand).astype(jnp.int32), axis=1, keepdims=True)
        t = jnp.where(cnt >= k, cand, t)
        for b in range(30, -1, -1):
            cand = t | jnp.int32(1 << b)
            cnt = jnp.sum((key >= cand).astype(jnp.int32), axis=1, keepdims=True)
            t = jnp.where(cnt >= k, cand, t)
        masks.append((key >= t).astype(jnp.float32))
    o_ref[...] = jnp.stack(masks, axis=1)


def _mask_from_t_kernel(s_ref, g_ref, t_ref, o_ref):
    key = _key_of(s_ref[...] + g_ref[...])
    masks = []
    for j in range(len(_KS)):
        masks.append((key >= t_ref[:, j:j + 1]).astype(jnp.float32))
    o_ref[...] = jnp.stack(masks, axis=1)


_NBINS = 4096
_NCHUNK = _N // 16


def _sc_select(s_hbm, g_hbm, out_hbm, s_v, g_v, key_v, hist_v, sarr_v,
               surv0, surv1, surv2, surv3, t_v):
    wid = lax.axis_index("s") * 2 + lax.axis_index("c")
    lanes = lax.broadcasted_iota(jnp.int32, (16,), 0)
    zero16 = jnp.zeros((16,), jnp.int32)
    ones16 = jnp.ones((16,), jnp.int32)
    survs = (surv0, surv1, surv2, surv3)

    def row_body(r, _):
        row = wid * _RPW + r
        pltpu.sync_copy(s_hbm.at[row], s_v)
        pltpu.sync_copy(g_hbm.at[row], g_v)

        def clear(j, _c):
            sl = pl.ds(j * 16, 16)
            hist_v[sl] = zero16
            key_v[sl] = _key_of(s_v[sl] + g_v[sl])
            return _c

        lax.fori_loop(0, _NCHUNK, clear, 0)

        def hist_body(j, _c):
            kv = key_v[pl.ds(j * 16, 16)]
            b = lax.shift_right_logical(kv ^ _I32_MIN, 20)
            plsc.addupdate_scatter(hist_v, [b], ones16)
            return _c

        lax.fori_loop(0, _NCHUNK, hist_body, 0)

        def suff_body(j, carry):
            jj = _NCHUNK - 1 - j
            sl = pl.ds(jj * 16, 16)
            h = lax.rev(hist_v[sl], (0,))
            s = lax.rev(plsc.cumsum(h), (0,)) + carry
            sarr_v[sl] = s
            return jnp.max(s) + zero16

        lax.fori_loop(0, _NCHUNK, suff_body, zero16)

        def s_at(idx):
            return sarr_v[pl.ds(idx, 16)][0]

        bks = []
        for k in _KS:
            b = jnp.int32(0)
            for step in (2048, 1024, 512, 256, 128, 64, 32, 16, 8, 4, 2, 1):
                nb = b + step
                ok = jnp.logical_and(nb <= _NBINS - 1, s_at(nb) >= k)
                b = jnp.where(ok, nb, b)
            bks.append(b)

        bsplat = [bk + zero16 for bk in bks]

        def comp_body(j, offs):
            kv = key_v[pl.ds(j * 16, 16)]
            b = lax.shift_right_logical(kv ^ _I32_MIN, 20)
            new = []
            for i in range(4):
                m = b == bsplat[i]
                plsc.store_compressed(survs[i].at[pl.ds(offs[i], 16)], kv,
                                      mask=m)
                new.append(offs[i] +
                           jnp.max(plsc.all_reduce_population_count(m)))
            return tuple(new)

        z = jnp.int32(0)
        lax.fori_loop(0, _NCHUNK, comp_body, (z, z, z, z))

        tks = []
        for i, k in enumerate(_KS):
            bk = bks[i]
            n_gt = jnp.where(bk >= _NBINS - 1, 0, s_at(bk + 1))
            rank = k - n_gt
            m_cnt = s_at(bk) - n_gt
            nch = lax.shift_right_logical(m_cnt + 15, 4)
            base_u = lax.shift_left(bk, 20)

            def count_ge(cand_signed, nch=nch, m_cnt=m_cnt, surv=survs[i]):
                cs = cand_signed + zero16

                def cbody(j, acc):
                    kv = surv[pl.ds(j * 16, 16)]
                    valid = (j * 16 + lanes) < m_cnt
                    hit = jnp.logical_and(valid, kv >= cs)
                    return acc + plsc.all_reduce_population_count(hit)

                return jnp.max(lax.fori_loop(0, nch, cbody, zero16))

            low = jnp.int32(0)
            for bit in range(19, -1, -1):
                cand = low | (1 << bit)
                c_signed = (base_u | cand) ^ _I32_MIN
                cnt = count_ge(c_signed)
                low = jnp.where(cnt >= rank, cand, low)
            tks.append((base_u | low) ^ _I32_MIN)

        tv = jnp.where(lanes == 0, tks[0],
             jnp.where(lanes == 1, tks[1],
             jnp.where(lanes == 2, tks[2],
             jnp.where(lanes == 3, tks[3], 0))))
        t_v[pl.ds(r * 16, 16)] = tv
        return _

    lax.fori_loop(0, _RPW, row_body, 0)
    pltpu.sync_copy(t_v, out_hbm.at[pl.ds(wid * _RPW * 16, _RPW * 16)])


_sc_select_call = functools.partial(
    pl.kernel,
    out_type=jax.ShapeDtypeStruct((_SC_ROWS * 16,), jnp.int32),
    scratch_types=[
        pltpu.VMEM((_N,), jnp.float32),
        pltpu.VMEM((_N,), jnp.float32),
        pltpu.VMEM((_N,), jnp.int32),
        pltpu.VMEM((_NBINS,), jnp.int32),
        pltpu.VMEM((_NBINS + 32,), jnp.int32),
        pltpu.VMEM((_N + 16,), jnp.int32),
        pltpu.VMEM((_N + 16,), jnp.int32),
        pltpu.VMEM((_N + 16,), jnp.int32),
        pltpu.VMEM((_N + 16,), jnp.int32),
        pltpu.VMEM((_RPW * 16,), jnp.int32),
    ],
    mesh=plsc.VectorSubcoreMesh(core_axis_name="c", subcore_axis_name="s"),
    compiler_params=pltpu.CompilerParams(needs_layout_passes=False),
)(_sc_select)


def kernel(scores):
    s2 = scores.reshape(_ROWS, _N)
    g2 = _GUMBEL

    s_tc, g_tc = s2[:_TC_ROWS], g2[:_TC_ROWS]
    s_sc, g_sc = s2[_TC_ROWS:], g2[_TC_ROWS:]

    t_flat = _sc_select_call(s_sc, g_sc)
    t_sc = t_flat.reshape(_SC_ROWS, 16)[:, :len(_KS)]

    out_tc = pl.pallas_call(
        _topk_mask_kernel,
        grid=(_TC_ROWS // _BLK,),
        in_specs=[
            pl.BlockSpec((_BLK, _N), lambda i: (i, 0)),
            pl.BlockSpec((_BLK, _N), lambda i: (i, 0)),
        ],
        out_specs=pl.BlockSpec((_BLK, len(_KS), _N), lambda i: (i, 0, 0)),
        out_shape=jax.ShapeDtypeStruct((_TC_ROWS, len(_KS), _N), jnp.float32),
    )(s_tc, g_tc)

    out_sc = pl.pallas_call(
        _mask_from_t_kernel,
        grid=(_SC_ROWS // _BLK,),
        in_specs=[
            pl.BlockSpec((_BLK, _N), lambda i: (i, 0)),
            pl.BlockSpec((_BLK, _N), lambda i: (i, 0)),
            pl.BlockSpec((_BLK, len(_KS)), lambda i: (i, 0)),
        ],
        out_specs=pl.BlockSpec((_BLK, len(_KS), _N), lambda i: (i, 0, 0)),
        out_shape=jax.ShapeDtypeStruct((_SC_ROWS, len(_KS), _N), jnp.float32),
    )(s_sc, g_sc, t_sc)

    out = jnp.concatenate([out_tc, out_sc], axis=0)
    return out.reshape(_B, _H, len(_KS), _N)


# speedup vs baseline: 1.9816x; 1.1018x over previous
import functools

import jax
import jax.numpy as jnp
from jax import lax
from jax.experimental import pallas as pl
from jax.experimental.pallas import tpu as pltpu
from jax.experimental.pallas import tpu_sc as plsc

_B, _H, _N = 64, 16, 4096
_KS = (16, 64, 256, 1024)
_ROWS = _B * _H
_BLK = 128
_SC_ROWS = 512
_TC_ROWS = _ROWS - _SC_ROWS
_NW = 32
_RPW = _SC_ROWS // _NW
_I32_MAX = 0x7FFFFFFF
_I32_MIN = -(2**31)


def _gumbel_const():
    u = jax.random.uniform(jax.random.key(42), (_B, _H, _N), dtype=jnp.float32)
    g = -jnp.log(-jnp.log(u + 1e-20) + 1e-20)
    return g.reshape(_ROWS, _N)


_GUMBEL = _gumbel_const()


def _key_of(p):
    bits = lax.bitcast_convert_type(p, jnp.int32)
    return jnp.where(bits < 0, bits ^ _I32_MAX, bits)


def _topk_mask_kernel(s_ref, g_ref, o_ref):
    key = _key_of(s_ref[...] + g_ref[...])
    r = key.shape[0]
    masks = []
    for k in _KS:
        t = jnp.full((r, 1), _I32_MIN, dtype=jnp.int32)
        cand = t & _I32_MAX
        cnt = jnp.sum((key >= cand).astype(jnp.int32), axis=1, keepdims=True)
        t = jnp.where(cnt >= k, cand, t)
        for b in range(30, -1, -1):
            cand = t | jnp.int32(1 << b)
            cnt = jnp.sum((key >= cand).astype(jnp.int32), axis=1, keepdims=True)
            t = jnp.where(cnt >= k, cand, t)
        masks.append((key >= t).astype(jnp.float32))
    o_ref[...] = jnp.stack(masks, axis=1)


def _mask_from_t_kernel(s_ref, g_ref, t_ref, o_ref):
    key = _key_of(s_ref[...] + g_ref[...])
    masks = []
    for j in range(len(_KS)):
        masks.append((key >= t_ref[:, j:j + 1]).astype(jnp.float32))
    o_ref[...] = jnp.stack(masks, axis=1)


_NBINS = 4096
_NCHUNK = _N // 16


def _sc_select(s_hbm, g_hbm, out_hbm, s_v, g_v, key_v, hist_v, sarr_v,
               surv0, surv1, surv2, surv3, t_v):
    wid = lax.axis_index("s") * 2 + lax.axis_index("c")
    lanes = lax.broadcasted_iota(jnp.int32, (16,), 0)
    zero16 = jnp.zeros((16,), jnp.int32)
    ones16 = jnp.ones((16,), jnp.int32)
    survs = (surv0, surv1, surv2, surv3)

    def row_body(r, _):
        row = wid * _RPW + r
        pltpu.sync_copy(s_hbm.at[row], s_v)
        pltpu.sync_copy(g_hbm.at[row], g_v)

        def clear(j, _c):
            sl = pl.ds(j * 16, 16)
            hist_v[sl] = zero16
            key_v[sl] = _key_of(s_v[sl] + g_v[sl])
            return _c

        lax.fori_loop(0, _NCHUNK, clear, 0)

        def hist_body(j, _c):
            kv = key_v[pl.ds(j * 16, 16)]
            b = lax.shift_right_logical(kv ^ _I32_MIN, 20)
            plsc.addupdate_scatter(hist_v, [b], ones16)
            return _c

        lax.fori_loop(0, _NCHUNK, hist_body, 0)

        def suff_body(j, carry):
            jj = _NCHUNK - 1 - j
            sl = pl.ds(jj * 16, 16)
            h = lax.rev(hist_v[sl], (0,))
            s = lax.rev(plsc.cumsum(h), (0,)) + carry
            sarr_v[sl] = s
            return s[0] + zero16

        lax.fori_loop(0, _NCHUNK, suff_body, zero16)

        def s_at(idx):
            return sarr_v[pl.ds(idx, 16)][0]

        bks = []
        for k in _KS:
            b = jnp.int32(0)
            for step in (2048, 1024, 512, 256, 128, 64, 32, 16, 8, 4, 2, 1):
                nb = b + step
                ok = jnp.logical_and(nb <= _NBINS - 1, s_at(nb) >= k)
                b = jnp.where(ok, nb, b)
            bks.append(b)

        bsplat = [bk + zero16 for bk in bks]

        def comp_body(j, offs):
            kv = key_v[pl.ds(j * 16, 16)]
            b = lax.shift_right_logical(kv ^ _I32_MIN, 20)
            new = []
            for i in range(4):
                m = b == bsplat[i]
                plsc.store_compressed(survs[i].at[pl.ds(offs[i], 16)], kv,
                                      mask=m)
                new.append(offs[i] +
                           plsc.all_reduce_population_count(m)[0])
            return tuple(new)

        z = jnp.int32(0)
        lax.fori_loop(0, _NCHUNK, comp_body, (z, z, z, z))

        tks = []
        for i, k in enumerate(_KS):
            bk = bks[i]
            n_gt = jnp.where(bk >= _NBINS - 1, 0, s_at(bk + 1))
            rank = k - n_gt
            m_cnt = s_at(bk) - n_gt
            nch = lax.shift_right_logical(m_cnt + 15, 4)
            base_u = lax.shift_left(bk, 20)

            def count_ge(cand_signed, nch=nch, m_cnt=m_cnt, surv=survs[i]):
                cs = cand_signed + zero16

                def cbody(j, acc):
                    kv = surv[pl.ds(j * 16, 16)]
                    valid = (j * 16 + lanes) < m_cnt
                    hit = jnp.logical_and(valid, kv >= cs)
                    return acc + plsc.all_reduce_population_count(hit)

                return lax.fori_loop(0, nch, cbody, zero16)[0]

            low = jnp.int32(0)
            for bit in range(19, -1, -1):
                cand = low | (1 << bit)
                c_signed = (base_u | cand) ^ _I32_MIN
                cnt = count_ge(c_signed)
                low = jnp.where(cnt >= rank, cand, low)
            tks.append((base_u | low) ^ _I32_MIN)

        tv = jnp.where(lanes == 0, tks[0],
             jnp.where(lanes == 1, tks[1],
             jnp.where(lanes == 2, tks[2],
             jnp.where(lanes == 3, tks[3], 0))))
        t_v[pl.ds(r * 16, 16)] = tv
        return _

    lax.fori_loop(0, _RPW, row_body, 0)
    pltpu.sync_copy(t_v, out_hbm.at[pl.ds(wid * _RPW * 16, _RPW * 16)])


_sc_select_call = functools.partial(
    pl.kernel,
    out_type=jax.ShapeDtypeStruct((_SC_ROWS * 16,), jnp.int32),
    scratch_types=[
        pltpu.VMEM((_N,), jnp.float32),
        pltpu.VMEM((_N,), jnp.float32),
        pltpu.VMEM((_N,), jnp.int32),
        pltpu.VMEM((_NBINS,), jnp.int32),
        pltpu.VMEM((_NBINS + 32,), jnp.int32),
        pltpu.VMEM((_N + 16,), jnp.int32),
        pltpu.VMEM((_N + 16,), jnp.int32),
        pltpu.VMEM((_N + 16,), jnp.int32),
        pltpu.VMEM((_N + 16,), jnp.int32),
        pltpu.VMEM((_RPW * 16,), jnp.int32),
    ],
    mesh=plsc.VectorSubcoreMesh(core_axis_name="c", subcore_axis_name="s"),
    compiler_params=pltpu.CompilerParams(needs_layout_passes=False),
)(_sc_select)


def kernel(scores):
    s2 = scores.reshape(_ROWS, _N)
    g2 = _GUMBEL

    s_tc, g_tc = s2[:_TC_ROWS], g2[:_TC_ROWS]
    s_sc, g_sc = s2[_TC_ROWS:], g2[_TC_ROWS:]

    t_flat = _sc_select_call(s_sc, g_sc)
    t_sc = t_flat.reshape(_SC_ROWS, 16)[:, :len(_KS)]

    out_tc = pl.pallas_call(
        _topk_mask_kernel,
        grid=(_TC_ROWS // _BLK,),
        in_specs=[
            pl.BlockSpec((_BLK, _N), lambda i: (i, 0)),
            pl.BlockSpec((_BLK, _N), lambda i: (i, 0)),
        ],
        out_specs=pl.BlockSpec((_BLK, len(_KS), _N), lambda i: (i, 0, 0)),
        out_shape=jax.ShapeDtypeStruct((_TC_ROWS, len(_KS), _N), jnp.float32),
    )(s_tc, g_tc)

    out_sc = pl.pallas_call(
        _mask_from_t_kernel,
        grid=(_SC_ROWS // _BLK,),
        in_specs=[
            pl.BlockSpec((_BLK, _N), lambda i: (i, 0)),
            pl.BlockSpec((_BLK, _N), lambda i: (i, 0)),
            pl.BlockSpec((_BLK, len(_KS)), lambda i: (i, 0)),
        ],
        out_specs=pl.BlockSpec((_BLK, len(_KS), _N), lambda i: (i, 0, 0)),
        out_shape=jax.ShapeDtypeStruct((_SC_ROWS, len(_KS), _N), jnp.float32),
    )(s_sc, g_sc, t_sc)

    out = jnp.concatenate([out_tc, out_sc], axis=0)
    return out.reshape(_B, _H, len(_KS), _N)


# speedup vs baseline: 2.0221x; 1.0204x over previous
import functools

import jax
import jax.numpy as jnp
from jax import lax
from jax.experimental import pallas as pl
from jax.experimental.pallas import tpu as pltpu
from jax.experimental.pallas import tpu_sc as plsc

_B, _H, _N = 64, 16, 4096
_KS = (16, 64, 256, 1024)
_ROWS = _B * _H
_BLK = 128
_SC_ROWS = 448
_TC_ROWS = _ROWS - _SC_ROWS
_NW = 32
_RPW = _SC_ROWS // _NW
_I32_MAX = 0x7FFFFFFF
_I32_MIN = -(2**31)


def _gumbel_const():
    u = jax.random.uniform(jax.random.key(42), (_B, _H, _N), dtype=jnp.float32)
    g = -jnp.log(-jnp.log(u + 1e-20) + 1e-20)
    return g.reshape(_ROWS, _N)


_GUMBEL = _gumbel_const()


def _key_of(p):
    bits = lax.bitcast_convert_type(p, jnp.int32)
    return jnp.where(bits < 0, bits ^ _I32_MAX, bits)


def _topk_mask_kernel(s_ref, g_ref, o_ref):
    key = _key_of(s_ref[...] + g_ref[...])
    r = key.shape[0]
    masks = []
    for k in _KS:
        t = jnp.full((r, 1), _I32_MIN, dtype=jnp.int32)
        cand = t & _I32_MAX
        cnt = jnp.sum((key >= cand).astype(jnp.int32), axis=1, keepdims=True)
        t = jnp.where(cnt >= k, cand, t)
        for b in range(30, -1, -1):
            cand = t | jnp.int32(1 << b)
            cnt = jnp.sum((key >= cand).astype(jnp.int32), axis=1, keepdims=True)
            t = jnp.where(cnt >= k, cand, t)
        masks.append((key >= t).astype(jnp.float32))
    o_ref[...] = jnp.stack(masks, axis=1)


def _mask_from_t_kernel(s_ref, g_ref, t_ref, o_ref):
    key = _key_of(s_ref[...] + g_ref[...])
    masks = []
    for j in range(len(_KS)):
        masks.append((key >= t_ref[:, j:j + 1]).astype(jnp.float32))
    o_ref[...] = jnp.stack(masks, axis=1)


_NBINS = 4096
_NCHUNK = _N // 16


def _sc_select(s_hbm, g_hbm, out_hbm, s_v, g_v, key_v, hist_v, sarr_v,
               surv0, surv1, surv2, surv3, t_v):
    wid = lax.axis_index("s") * 2 + lax.axis_index("c")
    lanes = lax.broadcasted_iota(jnp.int32, (16,), 0)
    zero16 = jnp.zeros((16,), jnp.int32)
    ones16 = jnp.ones((16,), jnp.int32)
    survs = (surv0, surv1, surv2, surv3)

    def row_body(r, _):
        row = wid * _RPW + r
        pltpu.sync_copy(s_hbm.at[row], s_v)
        pltpu.sync_copy(g_hbm.at[row], g_v)

        def clear(j, _c):
            sl = pl.ds(j * 16, 16)
            hist_v[sl] = zero16
            key_v[sl] = _key_of(s_v[sl] + g_v[sl])
            return _c

        lax.fori_loop(0, _NCHUNK, clear, 0)

        def hist_body(j, _c):
            kv = key_v[pl.ds(j * 16, 16)]
            b = lax.shift_right_logical(kv ^ _I32_MIN, 20)
            plsc.addupdate_scatter(hist_v, [b], ones16)
            return _c

        lax.fori_loop(0, _NCHUNK, hist_body, 0)

        def suff_body(j, carry):
            jj = _NCHUNK - 1 - j
            sl = pl.ds(jj * 16, 16)
            h = lax.rev(hist_v[sl], (0,))
            s = lax.rev(plsc.cumsum(h), (0,)) + carry
            sarr_v[sl] = s
            return s[0] + zero16

        lax.fori_loop(0, _NCHUNK, suff_body, zero16)

        def s_at(idx):
            return sarr_v[pl.ds(idx, 16)][0]

        bks = []
        for k in _KS:
            b = jnp.int32(0)
            for step in (2048, 1024, 512, 256, 128, 64, 32, 16, 8, 4, 2, 1):
                nb = b + step
                ok = jnp.logical_and(nb <= _NBINS - 1, s_at(nb) >= k)
                b = jnp.where(ok, nb, b)
            bks.append(b)

        bsplat = [bk + zero16 for bk in bks]

        def comp_body(j, offs):
            kv = key_v[pl.ds(j * 16, 16)]
            b = lax.shift_right_logical(kv ^ _I32_MIN, 20)
            new = []
            for i in range(4):
                m = b == bsplat[i]
                plsc.store_compressed(survs[i].at[pl.ds(offs[i], 16)], kv,
                                      mask=m)
                new.append(offs[i] +
                           plsc.all_reduce_population_count(m)[0])
            return tuple(new)

        z = jnp.int32(0)
        lax.fori_loop(0, _NCHUNK, comp_body, (z, z, z, z))

        tks = []
        for i, k in enumerate(_KS):
            bk = bks[i]
            n_gt = jnp.where(bk >= _NBINS - 1, 0, s_at(bk + 1))
            rank = k - n_gt
            m_cnt = s_at(bk) - n_gt
            nch = lax.shift_right_logical(m_cnt + 15, 4)
            base_u = lax.shift_left(bk, 20)

            def count_ge(cand_signed, nch=nch, m_cnt=m_cnt, surv=survs[i]):
                cs = cand_signed + zero16

                def cbody(j, acc):
                    kv = surv[pl.ds(j * 16, 16)]
                    valid = (j * 16 + lanes) < m_cnt
                    hit = jnp.logical_and(valid, kv >= cs)
                    return acc + plsc.all_reduce_population_count(hit)

                return lax.fori_loop(0, nch, cbody, zero16)[0]

            low = jnp.int32(0)
            for bit in range(19, -1, -1):
                cand = low | (1 << bit)
                c_signed = (base_u | cand) ^ _I32_MIN
                cnt = count_ge(c_signed)
                low = jnp.where(cnt >= rank, cand, low)
            tks.append((base_u | low) ^ _I32_MIN)

        tv = jnp.where(lanes == 0, tks[0],
             jnp.where(lanes == 1, tks[1],
             jnp.where(lanes == 2, tks[2],
             jnp.where(lanes == 3, tks[3], 0))))
        t_v[pl.ds(r * 16, 16)] = tv
        return _

    lax.fori_loop(0, _RPW, row_body, 0)
    pltpu.sync_copy(t_v, out_hbm.at[pl.ds(wid * _RPW * 16, _RPW * 16)])


_sc_select_call = functools.partial(
    pl.kernel,
    out_type=jax.ShapeDtypeStruct((_SC_ROWS * 16,), jnp.int32),
    scratch_types=[
        pltpu.VMEM((_N,), jnp.float32),
        pltpu.VMEM((_N,), jnp.float32),
        pltpu.VMEM((_N,), jnp.int32),
        pltpu.VMEM((_NBINS,), jnp.int32),
        pltpu.VMEM((_NBINS + 32,), jnp.int32),
        pltpu.VMEM((_N + 16,), jnp.int32),
        pltpu.VMEM((_N + 16,), jnp.int32),
        pltpu.VMEM((_N + 16,), jnp.int32),
        pltpu.VMEM((_N + 16,), jnp.int32),
        pltpu.VMEM((_RPW * 16,), jnp.int32),
    ],
    mesh=plsc.VectorSubcoreMesh(core_axis_name="c", subcore_axis_name="s"),
    compiler_params=pltpu.CompilerParams(needs_layout_passes=False),
)(_sc_select)


def kernel(scores):
    s2 = scores.reshape(_ROWS, _N)
    g2 = _GUMBEL

    s_tc, g_tc = s2[:_TC_ROWS], g2[:_TC_ROWS]
    s_sc, g_sc = s2[_TC_ROWS:], g2[_TC_ROWS:]

    t_flat = _sc_select_call(s_sc, g_sc)
    t_sc = t_flat.reshape(_SC_ROWS, 16)[:, :len(_KS)]

    out_tc = pl.pallas_call(
        _topk_mask_kernel,
        grid=(_TC_ROWS // _BLK,),
        in_specs=[
            pl.BlockSpec((_BLK, _N), lambda i: (i, 0)),
            pl.BlockSpec((_BLK, _N), lambda i: (i, 0)),
        ],
        out_specs=pl.BlockSpec((_BLK, len(_KS), _N), lambda i: (i, 0, 0)),
        out_shape=jax.ShapeDtypeStruct((_TC_ROWS, len(_KS), _N), jnp.float32),
    )(s_tc, g_tc)

    out_sc = pl.pallas_call(
        _mask_from_t_kernel,
        grid=(_SC_ROWS // _BLK,),
        in_specs=[
            pl.BlockSpec((_BLK, _N), lambda i: (i, 0)),
            pl.BlockSpec((_BLK, _N), lambda i: (i, 0)),
            pl.BlockSpec((_BLK, len(_KS)), lambda i: (i, 0)),
        ],
        out_specs=pl.BlockSpec((_BLK, len(_KS), _N), lambda i: (i, 0, 0)),
        out_shape=jax.ShapeDtypeStruct((_SC_ROWS, len(_KS), _N), jnp.float32),
    )(s_sc, g_sc, t_sc)

    out = jnp.concatenate([out_tc, out_sc], axis=0)
    return out.reshape(_B, _H, len(_KS), _N)


# speedup vs baseline: 2.3227x; 1.1487x over previous
import functools

import jax
import jax.numpy as jnp
from jax import lax
from jax.experimental import pallas as pl
from jax.experimental.pallas import tpu as pltpu
from jax.experimental.pallas import tpu_sc as plsc

_B, _H, _N = 64, 16, 4096
_KS = (16, 64, 256, 1024)
_ROWS = _B * _H
_BLK = 64
_SC_ROWS = 448
_TC_ROWS = _ROWS - _SC_ROWS
_NW = 32
_RPW = _SC_ROWS // _NW
_I32_MAX = 0x7FFFFFFF
_I32_MIN = -(2**31)


def _gumbel_const():
    u = jax.random.uniform(jax.random.key(42), (_B, _H, _N), dtype=jnp.float32)
    g = -jnp.log(-jnp.log(u + 1e-20) + 1e-20)
    return g.reshape(_ROWS, _N)


_GUMBEL = _gumbel_const()


def _key_of(p):
    bits = lax.bitcast_convert_type(p, jnp.int32)
    return jnp.where(bits < 0, bits ^ _I32_MAX, bits)


def _topk_mask_kernel(s_ref, g_ref, o_ref):
    key = _key_of(s_ref[...] + g_ref[...])
    r = key.shape[0]
    masks = []
    for k in _KS:
        t = jnp.full((r, 1), _I32_MIN, dtype=jnp.int32)
        cand = t & _I32_MAX
        cnt = jnp.sum((key >= cand).astype(jnp.int32), axis=1, keepdims=True)
        t = jnp.where(cnt >= k, cand, t)
        for b in range(30, -1, -1):
            cand = t | jnp.int32(1 << b)
            cnt = jnp.sum((key >= cand).astype(jnp.int32), axis=1, keepdims=True)
            t = jnp.where(cnt >= k, cand, t)
        masks.append((key >= t).astype(jnp.float32))
    o_ref[...] = jnp.stack(masks, axis=1)


def _mask_from_t_kernel(s_ref, g_ref, t_ref, o_ref):
    key = _key_of(s_ref[...] + g_ref[...])
    masks = []
    for j in range(len(_KS)):
        masks.append((key >= t_ref[:, j:j + 1]).astype(jnp.float32))
    o_ref[...] = jnp.stack(masks, axis=1)


_NBINS = 4096
_NCHUNK = _N // 16


def _sc_select(s_hbm, g_hbm, out_hbm, s_v, g_v, key_v, hist_v, sarr_v,
               surv0, surv1, surv2, surv3, t_v):
    wid = lax.axis_index("s") * 2 + lax.axis_index("c")
    lanes = lax.broadcasted_iota(jnp.int32, (16,), 0)
    zero16 = jnp.zeros((16,), jnp.int32)
    ones16 = jnp.ones((16,), jnp.int32)
    survs = (surv0, surv1, surv2, surv3)

    def row_body(r, _):
        row = wid * _RPW + r
        pltpu.sync_copy(s_hbm.at[row], s_v)
        pltpu.sync_copy(g_hbm.at[row], g_v)

        def clear(j, _c):
            sl = pl.ds(j * 16, 16)
            hist_v[sl] = zero16
            key_v[sl] = _key_of(s_v[sl] + g_v[sl])
            return _c

        lax.fori_loop(0, _NCHUNK, clear, 0)

        def hist_body(j, _c):
            kv = key_v[pl.ds(j * 16, 16)]
            b = lax.shift_right_logical(kv ^ _I32_MIN, 20)
            plsc.addupdate_scatter(hist_v, [b], ones16)
            return _c

        lax.fori_loop(0, _NCHUNK, hist_body, 0)

        def suff_body(j, carry):
            jj = _NCHUNK - 1 - j
            sl = pl.ds(jj * 16, 16)
            h = lax.rev(hist_v[sl], (0,))
            s = lax.rev(plsc.cumsum(h), (0,)) + carry
            sarr_v[sl] = s
            return s[0] + zero16

        lax.fori_loop(0, _NCHUNK, suff_body, zero16)

        def s_at(idx):
            return sarr_v[pl.ds(idx, 16)][0]

        bks = []
        for k in _KS:
            b = jnp.int32(0)
            for step in (2048, 1024, 512, 256, 128, 64, 32, 16, 8, 4, 2, 1):
                nb = b + step
                ok = jnp.logical_and(nb <= _NBINS - 1, s_at(nb) >= k)
                b = jnp.where(ok, nb, b)
            bks.append(b)

        bsplat = [bk + zero16 for bk in bks]

        def comp_body(j, offs):
            kv = key_v[pl.ds(j * 16, 16)]
            b = lax.shift_right_logical(kv ^ _I32_MIN, 20)
            new = []
            for i in range(4):
                m = b == bsplat[i]
                plsc.store_compressed(survs[i].at[pl.ds(offs[i], 16)], kv,
                                      mask=m)
                new.append(offs[i] +
                           plsc.all_reduce_population_count(m)[0])
            return tuple(new)

        z = jnp.int32(0)
        lax.fori_loop(0, _NCHUNK, comp_body, (z, z, z, z))

        tks = []
        for i, k in enumerate(_KS):
            bk = bks[i]
            n_gt = jnp.where(bk >= _NBINS - 1, 0, s_at(bk + 1))
            rank = k - n_gt
            m_cnt = s_at(bk) - n_gt
            nch = lax.shift_right_logical(m_cnt + 15, 4)
            base_u = lax.shift_left(bk, 20)

            def count_ge(cand_signed, nch=nch, m_cnt=m_cnt, surv=survs[i]):
                cs = cand_signed + zero16

                def cbody(j, acc):
                    kv = surv[pl.ds(j * 16, 16)]
                    valid = (j * 16 + lanes) < m_cnt
                    hit = jnp.logical_and(valid, kv >= cs)
                    return acc + plsc.all_reduce_population_count(hit)

                return lax.fori_loop(0, nch, cbody, zero16)[0]

            low = jnp.int32(0)
            for bit in range(19, -1, -1):
                cand = low | (1 << bit)
                c_signed = (base_u | cand) ^ _I32_MIN
                cnt = count_ge(c_signed)
                low = jnp.where(cnt >= rank, cand, low)
            tks.append((base_u | low) ^ _I32_MIN)

        tv = jnp.where(lanes == 0, tks[0],
             jnp.where(lanes == 1, tks[1],
             jnp.where(lanes == 2, tks[2],
             jnp.where(lanes == 3, tks[3], 0))))
        t_v[pl.ds(r * 16, 16)] = tv
        return _

    lax.fori_loop(0, _RPW, row_body, 0)
    pltpu.sync_copy(t_v, out_hbm.at[pl.ds(wid * _RPW * 16, _RPW * 16)])


_sc_select_call = functools.partial(
    pl.kernel,
    out_type=jax.ShapeDtypeStruct((_SC_ROWS * 16,), jnp.int32),
    scratch_types=[
        pltpu.VMEM((_N,), jnp.float32),
        pltpu.VMEM((_N,), jnp.float32),
        pltpu.VMEM((_N,), jnp.int32),
        pltpu.VMEM((_NBINS,), jnp.int32),
        pltpu.VMEM((_NBINS + 32,), jnp.int32),
        pltpu.VMEM((_N + 16,), jnp.int32),
        pltpu.VMEM((_N + 16,), jnp.int32),
        pltpu.VMEM((_N + 16,), jnp.int32),
        pltpu.VMEM((_N + 16,), jnp.int32),
        pltpu.VMEM((_RPW * 16,), jnp.int32),
    ],
    mesh=plsc.VectorSubcoreMesh(core_axis_name="c", subcore_axis_name="s"),
    compiler_params=pltpu.CompilerParams(needs_layout_passes=False),
)(_sc_select)


def kernel(scores):
    s2 = scores.reshape(_ROWS, _N)
    g2 = _GUMBEL

    s_tc, g_tc = s2[:_TC_ROWS], g2[:_TC_ROWS]
    s_sc, g_sc = s2[_TC_ROWS:], g2[_TC_ROWS:]

    t_flat = _sc_select_call(s_sc, g_sc)
    t_sc = t_flat.reshape(_SC_ROWS, 16)[:, :len(_KS)]

    out_tc = pl.pallas_call(
        _topk_mask_kernel,
        grid=(_TC_ROWS // _BLK,),
        in_specs=[
            pl.BlockSpec((_BLK, _N), lambda i: (i, 0)),
            pl.BlockSpec((_BLK, _N), lambda i: (i, 0)),
        ],
        out_specs=pl.BlockSpec((_BLK, len(_KS), _N), lambda i: (i, 0, 0)),
        out_shape=jax.ShapeDtypeStruct((_TC_ROWS, len(_KS), _N), jnp.float32),
    )(s_tc, g_tc)

    out_sc = pl.pallas_call(
        _mask_from_t_kernel,
        grid=(_SC_ROWS // _BLK,),
        in_specs=[
            pl.BlockSpec((_BLK, _N), lambda i: (i, 0)),
            pl.BlockSpec((_BLK, _N), lambda i: (i, 0)),
            pl.BlockSpec((_BLK, len(_KS)), lambda i: (i, 0)),
        ],
        out_specs=pl.BlockSpec((_BLK, len(_KS), _N), lambda i: (i, 0, 0)),
        out_shape=jax.ShapeDtypeStruct((_SC_ROWS, len(_KS), _N), jnp.float32),
    )(s_sc, g_sc, t_sc)

    out = jnp.concatenate([out_tc, out_sc], axis=0)
    return out.reshape(_B, _H, len(_KS), _N)
